# speedup vs baseline: 1.2980x; 1.2980x over previous
import jax
import jax.numpy as jnp
from jax.experimental import pallas as pl
from jax.experimental.pallas import tpu as pltpu

_N, _D, _K = 262144, 64, 512
_BLK = 1024
_NB = _N // _BLK


def _vq_block(x_ref, y_ref, emb_ref, q_ref, sums_ref):
    i = pl.program_id(0)
    x = x_ref[...]
    y = y_ref[...]
    emb = emb_ref[...]
    xe = jax.lax.dot_general(x, emb, (((1,), (1,)), ((), ())),
                             precision=jax.lax.Precision.DEFAULT)
    xnorm = jnp.sum(x * x, axis=1, keepdims=True)
    enorm = jnp.sum(emb * emb, axis=1)
    dist = (xnorm + enorm[None, :]) - 2.0 * xe
    minval = jnp.min(dist, axis=1, keepdims=True)
    iota = jax.lax.broadcasted_iota(jnp.int32, dist.shape, 1)
    idxc = jnp.min(jnp.where(dist == minval, iota, _K), axis=1, keepdims=True)
    oh = jnp.where(iota == idxc, 1.0, 0.0)
    q = jax.lax.dot_general(oh, emb, (((1,), (0,)), ((), ())),
                            precision=jax.lax.Precision.HIGHEST)
    q_ref[...] = q
    s1 = jnp.sum((q - x) ** 2)
    s2 = jnp.sum((y - q) ** 2)

    @pl.when(i == 0)
    def _():
        sums_ref[0] = 0.0
        sums_ref[1] = 0.0

    sums_ref[0] += s1
    sums_ref[1] += s2


def kernel(x, y, embeddings):
    q, sums = pl.pallas_call(
        _vq_block,
        grid=(_NB,),
        in_specs=[
            pl.BlockSpec((_BLK, _D), lambda i: (i, 0)),
            pl.BlockSpec((_BLK, _D), lambda i: (i, 0)),
            pl.BlockSpec((_K, _D), lambda i: (0, 0)),
        ],
        out_specs=[
            pl.BlockSpec((_BLK, _D), lambda i: (i, 0)),
            pl.BlockSpec(block_shape=(2,), index_map=lambda i: (0,),
                         memory_space=pltpu.SMEM),
        ],
        out_shape=[
            jax.ShapeDtypeStruct((_N, _D), jnp.float32),
            jax.ShapeDtypeStruct((2,), jnp.float32),
        ],
    )(x, y, embeddings)
    loss = 1.25 * (sums[0] + sums[1]) / (_N * _D)
    return q, loss


# speedup vs baseline: 1.8903x; 1.4563x over previous
import jax
import jax.numpy as jnp
from jax.experimental import pallas as pl
from jax.experimental.pallas import tpu as pltpu

_N, _D, _K = 262144, 64, 512
_BLK = 2048
_NB = _N // _BLK


def _vq_block(x_ref, y_ref, emb_ref, q_ref, sums_ref):
    i = pl.program_id(0)
    x = x_ref[...]
    y = y_ref[...]
    emb = emb_ref[...]
    xe = jax.lax.dot_general(x, emb, (((1,), (1,)), ((), ())),
                             precision=jax.lax.Precision.DEFAULT)
    xnorm = jnp.sum(x * x, axis=1, keepdims=True)
    enorm = jnp.sum(emb * emb, axis=1)
    dist = (xnorm + enorm[None, :]) - 2.0 * xe
    minval = jnp.min(dist, axis=1, keepdims=True)
    iota = jax.lax.broadcasted_iota(jnp.int32, dist.shape, 1)
    idxc = jnp.min(jnp.where(dist == minval, iota, _K), axis=1, keepdims=True)
    oh = jnp.where(iota == idxc, 1.0, 0.0).astype(jnp.bfloat16)
    emb_hi = emb.astype(jnp.bfloat16)
    emb_lo = (emb - emb_hi.astype(jnp.float32)).astype(jnp.bfloat16)
    q = (jax.lax.dot_general(oh, emb_hi, (((1,), (0,)), ((), ())),
                             preferred_element_type=jnp.float32)
         + jax.lax.dot_general(oh, emb_lo, (((1,), (0,)), ((), ())),
                               preferred_element_type=jnp.float32))
    q_ref[...] = q
    s1 = jnp.sum(minval)
    s2 = jnp.sum((y - q) ** 2)

    @pl.when(i == 0)
    def _():
        sums_ref[0] = 0.0
        sums_ref[1] = 0.0

    sums_ref[0] += s1
    sums_ref[1] += s2


def kernel(x, y, embeddings):
    q, sums = pl.pallas_call(
        _vq_block,
        grid=(_NB,),
        in_specs=[
            pl.BlockSpec((_BLK, _D), lambda i: (i, 0)),
            pl.BlockSpec((_BLK, _D), lambda i: (i, 0)),
            pl.BlockSpec((_K, _D), lambda i: (0, 0)),
        ],
        out_specs=[
            pl.BlockSpec((_BLK, _D), lambda i: (i, 0)),
            pl.BlockSpec(block_shape=(2,), index_map=lambda i: (0,),
                         memory_space=pltpu.SMEM),
        ],
        out_shape=[
            jax.ShapeDtypeStruct((_N, _D), jnp.float32),
            jax.ShapeDtypeStruct((2,), jnp.float32),
        ],
    )(x, y, embeddings)
    loss = 1.25 * (sums[0] + sums[1]) / (_N * _D)
    return q, loss
